# baseline (speedup 1.0000x reference)
import dataclasses
import functools

import jax
import jax.numpy as jnp
from jax import lax
from jax.experimental import pallas as pl
from jax.experimental.pallas import tpu as pltpu
from jax.experimental.pallas import tpu_sc as plsc

W, H, D = 32, 32, 32
G = W * H * D
N, C = 100000, 128

NC, NS = 2, 16
NW = NC * NS
NP = 102400
SL = NP // NW
CH = 2048
NCHUNK = NP // CH
GB = 512
NRANGE = G // GB
B = 128

_mesh = plsc.VectorSubcoreMesh(core_axis_name="c", subcore_axis_name="s")


def _compiler_params():
    cp = pltpu.CompilerParams()
    if "needs_layout_passes" in pltpu.CompilerParams.__dataclass_fields__:
        cp = dataclasses.replace(cp, needs_layout_passes=False)
    return cp


_GDN = lax.GatherDimensionNumbers(
    offset_dims=(), collapsed_slice_dims=(0,), start_index_map=(0,))


def _bcast_lane(vec, lane):
    idx = jnp.broadcast_to(lane.astype(jnp.int32), (16,))[:, None]
    return lax.gather(vec, idx, _GDN, (1,),
                      mode=lax.GatherScatterMode.PROMISE_IN_BOUNDS)


@functools.partial(
    pl.kernel,
    out_type=jax.ShapeDtypeStruct((NP,), jnp.int32),
    mesh=_mesh,
    scratch_types=[
        pltpu.VMEM((3, SL), jnp.float32),
        pltpu.VMEM((SL,), jnp.int32),
    ],
    compiler_params=_compiler_params(),
)
def _voxelize(pts_hbm, cells_hbm, pbuf, cbuf):
    wid = lax.axis_index("s") * NC + lax.axis_index("c")
    base = wid * SL
    pltpu.sync_copy(pts_hbm.at[:, pl.ds(base, SL)], pbuf)
    iota = lax.iota(jnp.int32, 16)

    @pl.loop(0, SL // 16)
    def _(v):
        off = v * 16
        x = pbuf[0, pl.ds(off, 16)]
        y = pbuf[1, pl.ds(off, 16)]
        z = pbuf[2, pl.ds(off, 16)]
        ix = jnp.clip((x * W).astype(jnp.int32), 0, W - 1)
        iy = jnp.clip((y * H).astype(jnp.int32), 0, H - 1)
        iz = jnp.clip((z * D).astype(jnp.int32), 0, D - 1)
        flat = (ix * H + iy) * D + iz
        pid = base + off + iota
        cbuf[pl.ds(off, 16)] = jnp.where(pid < N, flat, -1)

    pltpu.sync_copy(cbuf, cells_hbm.at[pl.ds(base, SL)])


@functools.partial(
    pl.kernel,
    out_type=jax.ShapeDtypeStruct((G * C,), jnp.float32),
    mesh=_mesh,
    scratch_types=[
        pltpu.VMEM_SHARED((NP,), jnp.int32),
        pltpu.VMEM((CH,), jnp.int32),
        pltpu.VMEM((B,), jnp.int32),
        pltpu.VMEM((B,), jnp.int32),
        pltpu.VMEM((B, C), jnp.float32),
        pltpu.VMEM((GB * C,), jnp.float32),
    ],
    compiler_params=_compiler_params(),
)
def _pool(cells_hbm, feat_hbm, out_hbm, spc, cchunk, mpid, mcell, rows, acc):
    cid = lax.axis_index("c")
    sid = lax.axis_index("s")
    wid = sid * NC + cid

    @pl.when(sid == 0)
    def _():
        pltpu.sync_copy(cells_hbm, spc)
    plsc.subcore_barrier()

    iota = lax.iota(jnp.int32, 16)
    zeros16 = jnp.zeros((16,), jnp.float32)

    @pl.loop(0, B // 16)
    def _(i):
        mpid[pl.ds(i * 16, 16)] = jnp.zeros((16,), jnp.int32)
        mcell[pl.ds(i * 16, 16)] = jnp.zeros((16,), jnp.int32)

    for rnd in range(NRANGE // NW):
        base = (wid * (NRANGE // NW) + rnd) * GB

        @pl.loop(0, GB * C // 16)
        def _(i):
            acc[pl.ds(i * 16, 16)] = zeros16

        def flush(cnt):
            pltpu.sync_copy(feat_hbm.at[mpid], rows)

            def row_body(r, carry):
                grp = (r // 16) * 16
                lane = r - grp
                cv = mcell[pl.ds(grp, 16)]
                rbase = _bcast_lane(cv, lane) * C
                for j in range(C // 16):
                    offs = rbase + (j * 16) + iota
                    cur = plsc.load_gather(acc, [offs])
                    new = jnp.maximum(cur, rows[r, pl.ds(j * 16, 16)])
                    plsc.store_scatter(acc, [offs], new)
                return carry

            lax.fori_loop(0, cnt, row_body, 0)
            return jnp.int32(0)

        def chunk_body(k, cnt):
            pltpu.sync_copy(spc.at[pl.ds(k * CH, CH)], cchunk)

            def vec_body(v, cnt):
                cnt = lax.cond(cnt > B - 16, flush, lambda c: c, cnt)
                c = cchunk[pl.ds(v * 16, 16)]
                m = (c >= base) & (c < base + GB)
                pid = k * CH + v * 16 + iota
                plsc.store_compressed(mpid.at[pl.ds(cnt, 16)], pid, mask=m)
                plsc.store_compressed(mcell.at[pl.ds(cnt, 16)], c - base,
                                      mask=m)
                pc = plsc.all_reduce_population_count(m)
                return cnt + pc[0]

            return lax.fori_loop(0, CH // 16, vec_body, cnt)

        cnt = lax.fori_loop(0, NCHUNK, chunk_body, jnp.int32(0))
        flush(cnt)
        pltpu.sync_copy(acc, out_hbm.at[pl.ds(base * C, GB * C)])


def kernel(features, points):
    pts_t = jnp.zeros((3, NP), jnp.float32).at[:, :N].set(points.T)
    cells = _voxelize(pts_t)
    out_flat = _pool(cells, features)
    return out_flat.reshape(G, C)
